# Initial kernel scaffold; baseline (speedup 1.0000x reference)
import jax
import jax.numpy as jnp
from jax.experimental import pallas as pl


def kernel(x, table):
    raise NotImplementedError("write your pallas kernel here")


# speedup vs baseline: 2.7397x; 2.7397x over previous
import jax
import jax.numpy as jnp
from jax.experimental import pallas as pl

FEAT = 1024
ROW_BLOCK = 512


def _copy_body(t_ref, o_ref):
    o_ref[...] = t_ref[...]


def kernel(x, table):
    length = x.shape[1]
    grid = length // ROW_BLOCK
    return pl.pallas_call(
        _copy_body,
        grid=(grid,),
        in_specs=[pl.BlockSpec((ROW_BLOCK, FEAT), lambda i: (i, 0))],
        out_specs=pl.BlockSpec((ROW_BLOCK, FEAT), lambda i: (i, 0)),
        out_shape=jax.ShapeDtypeStruct((length, FEAT), table.dtype),
    )(table)
